# baseline (speedup 1.0000x reference)
import jax
import jax.numpy as jnp
from jax import lax
from jax.experimental import pallas as pl
from jax.experimental.pallas import tpu as pltpu
from jax.experimental.pallas import tpu_sc as plsc

N_ATOMS = 10000
N_PAIRS = 640000
NC = 2
NS = 16
NW = NC * NS
PER_TILE = N_PAIRS // NW
CHUNK = 4000
NCHUNK = PER_TILE // CHUNK
LANES = 16
UN = 10
UN2 = 25
NPAD = 10240


def _sc_pair_force(xs, ys, zs, pi, pj, out,
                   xb0, yb0, zb0, xb1, yb1, zb1, ib0, ib1, jb0, jb1,
                   accx, accy, accz, afx, afy, afz, ihead,
                   s00, s01, s02, s03, s04, s10, s11, s12, s13, s14, semh):
    c = lax.axis_index("c")
    s = lax.axis_index("s")
    wid = s * NC + c
    base0 = wid * PER_TILE
    zero16 = jnp.zeros((LANES,), jnp.float32)

    xbufs = (xb0, xb1)
    ybufs = (yb0, yb1)
    zbufs = (zb0, zb1)
    ibufs = (ib0, ib1)
    jbufs = (jb0, jb1)
    sems = ((s00, s01, s02, s03, s04), (s10, s11, s12, s13, s14))

    def start(ch):
        slot = ch % 2
        b = base0 + ch * CHUNK
        return (
            pltpu.async_copy(xs.at[pl.ds(b, CHUNK)], xbufs[slot], sems[slot][0]),
            pltpu.async_copy(ys.at[pl.ds(b, CHUNK)], ybufs[slot], sems[slot][1]),
            pltpu.async_copy(zs.at[pl.ds(b, CHUNK)], zbufs[slot], sems[slot][2]),
            pltpu.async_copy(pi.at[pl.ds(b, CHUNK)], ibufs[slot], sems[slot][3]),
            pltpu.async_copy(pj.at[pl.ds(b, CHUNK)], jbufs[slot], sems[slot][4]),
        )

    pending = {0: start(0)}
    hcp = pltpu.async_copy(pi.at[pl.ds(0, N_ATOMS)], ihead, semh)

    @plsc.parallel_loop(0, N_ATOMS // LANES, unroll=UN2)
    def zero_body(t):
        sl = pl.ds(t * LANES, LANES)
        accx[sl] = zero16
        accy[sl] = zero16
        accz[sl] = zero16
        afx[sl] = zero16
        afy[sl] = zero16
        afz[sl] = zero16

    for ch in range(NCHUNK):
        slot = ch % 2
        for h in pending.pop(ch):
            h.wait()
        if ch + 1 < NCHUNK:
            pending[ch + 1] = start(ch + 1)
        xbf = xbufs[slot]
        ybf = ybufs[slot]
        zbf = zbufs[slot]
        ibf = ibufs[slot]
        jbf = jbufs[slot]

        @plsc.parallel_loop(0, CHUNK // LANES, unroll=UN)
        def force_body(v):
            sl = pl.ds(v * LANES, LANES)
            x = xbf[sl]
            y = ybf[sl]
            z = zbf[sl]
            r2 = x * x + y * y + z * z + 0.01
            inv = 1.0 / r2
            inv3 = inv * inv * inv
            coef = inv3 * inv * (24.0 - 48.0 * inv3)
            fx = coef * x
            fy = coef * y
            fz = coef * z
            ii = ibf[sl]
            jj = jbf[sl]
            plsc.addupdate_scatter(accx, [ii], fx)
            plsc.addupdate_scatter(accy, [ii], fy)
            plsc.addupdate_scatter(accz, [ii], fz)
            plsc.addupdate_scatter(accx, [jj], -fx)
            plsc.addupdate_scatter(accy, [jj], -fy)
            plsc.addupdate_scatter(accz, [jj], -fz)

    hcp.wait()

    @plsc.parallel_loop(0, N_ATOMS // LANES, unroll=UN2)
    def stage2_body(t):
        sl = pl.ds(t * LANES, LANES)
        idx = ihead[sl]
        plsc.addupdate_scatter(afx, [idx], accx[sl])
        plsc.addupdate_scatter(afy, [idx], accy[sl])
        plsc.addupdate_scatter(afz, [idx], accz[sl])

    pltpu.sync_copy(afx, out.at[pl.ds(wid * NPAD, N_ATOMS)])
    pltpu.sync_copy(afy, out.at[pl.ds((NW + wid) * NPAD, N_ATOMS)])
    pltpu.sync_copy(afz, out.at[pl.ds((2 * NW + wid) * NPAD, N_ATOMS)])


def _combine_body(x_ref, o_ref):
    for comp in range(3):
        acc = x_ref[pl.ds(comp * NW * NPAD, NPAD)]
        for w in range(1, NW):
            acc = acc + x_ref[pl.ds((comp * NW + w) * NPAD, NPAD)]
        o_ref[comp, :] = acc


@jax.jit
def kernel(pair_dist, pair_i, pair_j, atom_batch):
    xs = pair_dist[:, 0]
    ys = pair_dist[:, 1]
    zs = pair_dist[:, 2]

    mesh = plsc.VectorSubcoreMesh(core_axis_name="c", subcore_axis_name="s")
    sc_fn = pl.kernel(
        _sc_pair_force,
        out_type=jax.ShapeDtypeStruct((3 * NW * NPAD,), jnp.float32),
        mesh=mesh,
        compiler_params=pltpu.CompilerParams(needs_layout_passes=False),
        scratch_types=[
            pltpu.VMEM((CHUNK,), jnp.float32),
            pltpu.VMEM((CHUNK,), jnp.float32),
            pltpu.VMEM((CHUNK,), jnp.float32),
            pltpu.VMEM((CHUNK,), jnp.float32),
            pltpu.VMEM((CHUNK,), jnp.float32),
            pltpu.VMEM((CHUNK,), jnp.float32),
            pltpu.VMEM((CHUNK,), jnp.int32),
            pltpu.VMEM((CHUNK,), jnp.int32),
            pltpu.VMEM((CHUNK,), jnp.int32),
            pltpu.VMEM((CHUNK,), jnp.int32),
            pltpu.VMEM((N_ATOMS,), jnp.float32),
            pltpu.VMEM((N_ATOMS,), jnp.float32),
            pltpu.VMEM((N_ATOMS,), jnp.float32),
            pltpu.VMEM((N_ATOMS,), jnp.float32),
            pltpu.VMEM((N_ATOMS,), jnp.float32),
            pltpu.VMEM((N_ATOMS,), jnp.float32),
            pltpu.VMEM((N_ATOMS,), jnp.int32),
            pltpu.SemaphoreType.DMA,
            pltpu.SemaphoreType.DMA,
            pltpu.SemaphoreType.DMA,
            pltpu.SemaphoreType.DMA,
            pltpu.SemaphoreType.DMA,
            pltpu.SemaphoreType.DMA,
            pltpu.SemaphoreType.DMA,
            pltpu.SemaphoreType.DMA,
            pltpu.SemaphoreType.DMA,
            pltpu.SemaphoreType.DMA,
            pltpu.SemaphoreType.DMA,
        ],
    )
    partials = sc_fn(xs, ys, zs, pair_i, pair_j)

    combined = pl.pallas_call(
        _combine_body,
        out_shape=jax.ShapeDtypeStruct((3, NPAD), jnp.float32),
    )(partials)

    return combined[:, :N_ATOMS].T


# speedup vs baseline: 1.0202x; 1.0202x over previous
import jax
import jax.numpy as jnp
from jax import lax
from jax.experimental import pallas as pl
from jax.experimental.pallas import tpu as pltpu
from jax.experimental.pallas import tpu_sc as plsc

N_ATOMS = 10000
N_PAIRS = 640000
NC = 2
NS = 16
NW = NC * NS
PER_TILE = N_PAIRS // NW
CHUNK = 4000
NCHUNK = PER_TILE // CHUNK
LANES = 16
UN = 5
UN2 = 25
NPAD = 10240


def _sc_pair_force(xs, ys, zs, pi, pj, out,
                   xb0, yb0, zb0, xb1, yb1, zb1, ib0, ib1, jb0, jb1,
                   accx, accy, accz, afx, afy, afz, ihead,
                   s00, s01, s02, s03, s04, s10, s11, s12, s13, s14, semh):
    c = lax.axis_index("c")
    s = lax.axis_index("s")
    wid = s * NC + c
    base0 = wid * PER_TILE
    zero16 = jnp.zeros((LANES,), jnp.float32)

    xbufs = (xb0, xb1)
    ybufs = (yb0, yb1)
    zbufs = (zb0, zb1)
    ibufs = (ib0, ib1)
    jbufs = (jb0, jb1)
    sems = ((s00, s01, s02, s03, s04), (s10, s11, s12, s13, s14))

    def start(ch):
        slot = ch % 2
        b = base0 + ch * CHUNK
        return (
            pltpu.async_copy(xs.at[pl.ds(b, CHUNK)], xbufs[slot], sems[slot][0]),
            pltpu.async_copy(ys.at[pl.ds(b, CHUNK)], ybufs[slot], sems[slot][1]),
            pltpu.async_copy(zs.at[pl.ds(b, CHUNK)], zbufs[slot], sems[slot][2]),
            pltpu.async_copy(pi.at[pl.ds(b, CHUNK)], ibufs[slot], sems[slot][3]),
            pltpu.async_copy(pj.at[pl.ds(b, CHUNK)], jbufs[slot], sems[slot][4]),
        )

    pending = {0: start(0)}
    hcp = pltpu.async_copy(pi.at[pl.ds(0, N_ATOMS)], ihead, semh)

    @plsc.parallel_loop(0, N_ATOMS // LANES, unroll=UN2)
    def zero_body(t):
        sl = pl.ds(t * LANES, LANES)
        accx[sl] = zero16
        accy[sl] = zero16
        accz[sl] = zero16
        afx[sl] = zero16
        afy[sl] = zero16
        afz[sl] = zero16

    for ch in range(NCHUNK):
        slot = ch % 2
        for h in pending.pop(ch):
            h.wait()
        if ch + 1 < NCHUNK:
            pending[ch + 1] = start(ch + 1)
        xbf = xbufs[slot]
        ybf = ybufs[slot]
        zbf = zbufs[slot]
        ibf = ibufs[slot]
        jbf = jbufs[slot]

        @plsc.parallel_loop(0, CHUNK // LANES, unroll=UN)
        def force_body(v):
            sl = pl.ds(v * LANES, LANES)
            x = xbf[sl]
            y = ybf[sl]
            z = zbf[sl]
            r2 = x * x + y * y + z * z + 0.01
            inv = 1.0 / r2
            inv3 = inv * inv * inv
            coef = inv3 * inv * (24.0 - 48.0 * inv3)
            fx = coef * x
            fy = coef * y
            fz = coef * z
            ii = ibf[sl]
            jj = jbf[sl]
            plsc.addupdate_scatter(accx, [ii], fx)
            plsc.addupdate_scatter(accy, [ii], fy)
            plsc.addupdate_scatter(accz, [ii], fz)
            plsc.addupdate_scatter(accx, [jj], -fx)
            plsc.addupdate_scatter(accy, [jj], -fy)
            plsc.addupdate_scatter(accz, [jj], -fz)

    hcp.wait()

    @plsc.parallel_loop(0, N_ATOMS // LANES, unroll=UN2)
    def stage2_body(t):
        sl = pl.ds(t * LANES, LANES)
        idx = ihead[sl]
        plsc.addupdate_scatter(afx, [idx], accx[sl])
        plsc.addupdate_scatter(afy, [idx], accy[sl])
        plsc.addupdate_scatter(afz, [idx], accz[sl])

    pltpu.sync_copy(afx, out.at[pl.ds(wid * NPAD, N_ATOMS)])
    pltpu.sync_copy(afy, out.at[pl.ds((NW + wid) * NPAD, N_ATOMS)])
    pltpu.sync_copy(afz, out.at[pl.ds((2 * NW + wid) * NPAD, N_ATOMS)])


def _combine_body(x_ref, o_ref):
    for comp in range(3):
        acc = x_ref[pl.ds(comp * NW * NPAD, NPAD)]
        for w in range(1, NW):
            acc = acc + x_ref[pl.ds((comp * NW + w) * NPAD, NPAD)]
        o_ref[comp, :] = acc


@jax.jit
def kernel(pair_dist, pair_i, pair_j, atom_batch):
    xs = pair_dist[:, 0]
    ys = pair_dist[:, 1]
    zs = pair_dist[:, 2]

    mesh = plsc.VectorSubcoreMesh(core_axis_name="c", subcore_axis_name="s")
    sc_fn = pl.kernel(
        _sc_pair_force,
        out_type=jax.ShapeDtypeStruct((3 * NW * NPAD,), jnp.float32),
        mesh=mesh,
        compiler_params=pltpu.CompilerParams(needs_layout_passes=False),
        scratch_types=[
            pltpu.VMEM((CHUNK,), jnp.float32),
            pltpu.VMEM((CHUNK,), jnp.float32),
            pltpu.VMEM((CHUNK,), jnp.float32),
            pltpu.VMEM((CHUNK,), jnp.float32),
            pltpu.VMEM((CHUNK,), jnp.float32),
            pltpu.VMEM((CHUNK,), jnp.float32),
            pltpu.VMEM((CHUNK,), jnp.int32),
            pltpu.VMEM((CHUNK,), jnp.int32),
            pltpu.VMEM((CHUNK,), jnp.int32),
            pltpu.VMEM((CHUNK,), jnp.int32),
            pltpu.VMEM((N_ATOMS,), jnp.float32),
            pltpu.VMEM((N_ATOMS,), jnp.float32),
            pltpu.VMEM((N_ATOMS,), jnp.float32),
            pltpu.VMEM((N_ATOMS,), jnp.float32),
            pltpu.VMEM((N_ATOMS,), jnp.float32),
            pltpu.VMEM((N_ATOMS,), jnp.float32),
            pltpu.VMEM((N_ATOMS,), jnp.int32),
            pltpu.SemaphoreType.DMA,
            pltpu.SemaphoreType.DMA,
            pltpu.SemaphoreType.DMA,
            pltpu.SemaphoreType.DMA,
            pltpu.SemaphoreType.DMA,
            pltpu.SemaphoreType.DMA,
            pltpu.SemaphoreType.DMA,
            pltpu.SemaphoreType.DMA,
            pltpu.SemaphoreType.DMA,
            pltpu.SemaphoreType.DMA,
            pltpu.SemaphoreType.DMA,
        ],
    )
    partials = sc_fn(xs, ys, zs, pair_i, pair_j)

    combined = pl.pallas_call(
        _combine_body,
        out_shape=jax.ShapeDtypeStruct((3, NPAD), jnp.float32),
    )(partials)

    return combined[:, :N_ATOMS].T
